# Initial kernel scaffold; baseline (speedup 1.0000x reference)
import functools

import jax
import jax.numpy as jnp
from jax import lax
from jax.experimental import pallas as pl
from jax.experimental.pallas import tpu as pltpu
from jax.experimental.pallas import tpu_sc as plsc

B, N = 4, 4096
C_FEAT = 64
FILTERS = 64
K_POINTS = 15
NSAMPLE = 16
CIN = C_FEAT + FILTERS
ROW = 144

BLKA = 128
TJ = 512
NT = N // TJ
BLKC = 128

_INF = jnp.float32(3.0e38)
_BIGI = jnp.int32(2**30)


def _knn_body(xyz_ref, xyzt_ref, cxyzt_ref, gidx_ref, nn_ref, d2_ref):
    b = pl.program_id(0)

    for t in range(NT):
        d2t = jnp.zeros((BLKA, TJ), jnp.float32)
        for c in range(3):
            xc = xyz_ref[0, :, c:c + 1]
            ac = xyzt_ref[0, c:c + 1, t * TJ:(t + 1) * TJ]
            d = xc - ac
            d2t = d2t + d * d
        d2_ref[:, t * TJ:(t + 1) * TJ] = d2t

    lv = jnp.full((BLKA, 1), -1.0, jnp.float32)
    li = jnp.full((BLKA, 1), -1, jnp.int32)
    cols = []
    for _ in range(NSAMPLE):
        m = jnp.full((BLKA, 1), _INF, jnp.float32)
        mi = jnp.full((BLKA, 1), _BIGI, jnp.int32)
        for t in range(NT):
            d2t = d2_ref[:, t * TJ:(t + 1) * TJ]
            iot = lax.broadcasted_iota(jnp.int32, (BLKA, TJ), 1) + t * TJ
            mask = (d2t > lv) | ((d2t == lv) & (iot > li))
            vals = jnp.where(mask, d2t, _INF)
            mt = jnp.min(vals, axis=1, keepdims=True)
            it = jnp.min(jnp.where(vals == mt, iot, _BIGI), axis=1,
                         keepdims=True)
            better = (mt < m) | ((mt == m) & (it < mi))
            m = jnp.where(better, mt, m)
            mi = jnp.where(better, it, mi)
        lv, li = m, mi
        cols.append(mi)
    gidx_ref[0] = jnp.concatenate(cols, axis=1) + b * N

    bm = jnp.full((BLKA, 1), _INF, jnp.float32)
    bi = jnp.full((BLKA, 1), _BIGI, jnp.int32)
    for t in range(NT):
        d2t = jnp.zeros((BLKA, TJ), jnp.float32)
        for c in range(3):
            xc = xyz_ref[0, :, c:c + 1]
            ac = cxyzt_ref[0, c:c + 1, t * TJ:(t + 1) * TJ]
            d = xc - ac
            d2t = d2t + d * d
        iot = lax.broadcasted_iota(jnp.int32, (BLKA, TJ), 1) + t * TJ
        mt = jnp.min(d2t, axis=1, keepdims=True)
        it = jnp.min(jnp.where(d2t == mt, iot, _BIGI), axis=1, keepdims=True)
        better = (mt < bm) | ((mt == bm) & (it < bi))
        bm = jnp.where(better, mt, bm)
        bi = jnp.where(better, it, bi)
    nn_ref[0] = bi + b * N


def _knn_call(xyz_p, xyz_t, cxyz_t):
    return pl.pallas_call(
        _knn_body,
        grid=(B, N // BLKA),
        in_specs=[
            pl.BlockSpec((1, BLKA, 4), lambda b, i: (b, i, 0)),
            pl.BlockSpec((1, 4, N), lambda b, i: (b, 0, 0)),
            pl.BlockSpec((1, 4, N), lambda b, i: (b, 0, 0)),
        ],
        out_specs=[
            pl.BlockSpec((1, BLKA, NSAMPLE), lambda b, i: (b, i, 0)),
            pl.BlockSpec((1, BLKA, 1), lambda b, i: (b, i, 0)),
        ],
        out_shape=[
            jax.ShapeDtypeStruct((B, N, NSAMPLE), jnp.int32),
            jax.ShapeDtypeStruct((B, N, 1), jnp.int32),
        ],
        scratch_shapes=[pltpu.VMEM((BLKA, N), jnp.float32)],
    )(xyz_p, xyz_t, cxyz_t)


_NW = 32
_CHUNK = 128
_FE = B * N * NSAMPLE
_FE_PER = _FE // _NW
_F_ITERS = _FE_PER // _CHUNK
_CH_PER = (B * N) // _NW
_C_ITERS = _CH_PER // _CHUNK


def _gather_body(tbl_hbm, gidx_hbm, c0f_hbm, nn_hbm, nf_out, ch_out,
                 idx_v, rows_v, rows2_v, sem):
    wid = lax.axis_index("s") * 2 + lax.axis_index("c")

    fbase = wid * _FE_PER

    def fbody(t, carry):
        off = fbase + t * _CHUNK
        pltpu.sync_copy(gidx_hbm.at[pl.ds(off, _CHUNK)], idx_v)
        pltpu.async_copy(tbl_hbm.at[idx_v], rows_v, sem).wait()
        pltpu.sync_copy(rows_v, nf_out.at[pl.ds(off, _CHUNK)])
        return carry

    lax.fori_loop(0, _F_ITERS, fbody, 0)

    cbase = wid * _CH_PER

    def cbody(t, carry):
        off = cbase + t * _CHUNK
        pltpu.sync_copy(nn_hbm.at[pl.ds(off, _CHUNK)], idx_v)
        pltpu.async_copy(c0f_hbm.at[idx_v], rows2_v, sem).wait()
        pltpu.sync_copy(rows2_v, ch_out.at[pl.ds(off, _CHUNK)])
        return carry

    lax.fori_loop(0, _C_ITERS, cbody, 0)


def _gather_call(tbl, gidx_f, c0f, nn_f):
    mesh = plsc.VectorSubcoreMesh(core_axis_name="c", subcore_axis_name="s")
    gk = functools.partial(
        pl.kernel,
        out_type=[
            jax.ShapeDtypeStruct((_FE, ROW), jnp.float32),
            jax.ShapeDtypeStruct((B * N, FILTERS), jnp.float32),
        ],
        mesh=mesh,
        scratch_types=[
            pltpu.VMEM((_CHUNK,), jnp.int32),
            pltpu.VMEM((_CHUNK, ROW), jnp.float32),
            pltpu.VMEM((_CHUNK, FILTERS), jnp.float32),
            pltpu.SemaphoreType.DMA,
        ],
    )(_gather_body)
    return gk(tbl, gidx_f, c0f, nn_f)


def _comp_body(nf_ref, xyz_ref, kp_ref, ch_ref, w_ref, hf_ref, cn_ref,
               wf_ref):
    xs = [xyz_ref[0, :, c:c + 1] for c in range(3)]
    for s in range(NSAMPLE):
        nrow = nf_ref[0, :, s, :]
        dist2 = jnp.full((BLKC, K_POINTS), 1e-12, jnp.float32)
        for c in range(3):
            rel = nrow[:, c:c + 1] - xs[c]
            d = rel - kp_ref[c:c + 1, :]
            dist2 = dist2 + d * d
        infl = jnp.maximum(1.0 - jnp.sqrt(dist2), 0.0)
        feat = nrow[:, 4:4 + CIN]
        for k in range(K_POINTS):
            w = infl[:, k:k + 1] * feat
            if s == 0:
                wf_ref[:, k * CIN:(k + 1) * CIN] = w
            else:
                wf_ref[:, k * CIN:(k + 1) * CIN] += w

    out = jnp.dot(wf_ref[...], w_ref[...],
                  preferred_element_type=jnp.float32)

    def hs(x):
        return jnp.clip(0.2 * x + 0.5, 0.0, 1.0)

    i_g = hs(out[:, 0:FILTERS])
    f_g = hs(out[:, FILTERS:2 * FILTERS])
    o_g = hs(out[:, 2 * FILTERS:3 * FILTERS])
    cc = out[:, 3 * FILTERS:4 * FILTERS]
    c_new = f_g * ch_ref[0] + i_g * jnp.tanh(cc)
    hf_ref[0] = o_g * jnp.tanh(c_new)
    cn_ref[0] = c_new


def _comp_call(nf, xyz_p, kp_t, ch, w_flat):
    return pl.pallas_call(
        _comp_body,
        grid=(B, N // BLKC),
        in_specs=[
            pl.BlockSpec((1, BLKC, NSAMPLE, ROW), lambda b, i: (b, i, 0, 0)),
            pl.BlockSpec((1, BLKC, 4), lambda b, i: (b, i, 0)),
            pl.BlockSpec((3, K_POINTS), lambda b, i: (0, 0)),
            pl.BlockSpec((1, BLKC, FILTERS), lambda b, i: (b, i, 0)),
            pl.BlockSpec((K_POINTS * CIN, 4 * FILTERS), lambda b, i: (0, 0)),
        ],
        out_specs=[
            pl.BlockSpec((1, BLKC, FILTERS), lambda b, i: (b, i, 0)),
            pl.BlockSpec((1, BLKC, FILTERS), lambda b, i: (b, i, 0)),
        ],
        out_shape=[
            jax.ShapeDtypeStruct((B, N, FILTERS), jnp.float32),
            jax.ShapeDtypeStruct((B, N, FILTERS), jnp.float32),
        ],
        scratch_shapes=[pltpu.VMEM((BLKC, K_POINTS * CIN), jnp.float32)],
    )(nf, xyz_p, kp_t, ch, w_flat)


def kernel(inputs, h0, c0, kernel_points, W_i, W_f, W_o, W_c):
    xyz = inputs[..., :3]
    feats = jnp.concatenate([h0[..., 3:], inputs[..., 3:]], axis=-1)

    zero1 = jnp.zeros((B, N, 1), jnp.float32)
    xyz_p = jnp.concatenate([xyz, zero1], axis=-1)
    xyz_t = jnp.concatenate(
        [jnp.transpose(xyz, (0, 2, 1)),
         jnp.zeros((B, 1, N), jnp.float32)], axis=1)
    cxyz_t = jnp.concatenate(
        [jnp.transpose(c0[..., :3], (0, 2, 1)),
         jnp.zeros((B, 1, N), jnp.float32)], axis=1)

    gidx, nn = _knn_call(xyz_p, xyz_t, cxyz_t)

    tbl = jnp.concatenate(
        [xyz_p, feats, jnp.zeros((B, N, ROW - 4 - CIN), jnp.float32)],
        axis=-1).reshape(B * N, ROW)
    c0f = c0[..., 3:].reshape(B * N, FILTERS)

    nf, ch = _gather_call(tbl, gidx.reshape(_FE), c0f, nn.reshape(B * N))
    nf = nf.reshape(B, N, NSAMPLE, ROW)
    ch = ch.reshape(B, N, FILTERS)

    kp_t = jnp.transpose(kernel_points, (1, 0))
    w_flat = jnp.concatenate(
        [W.reshape(K_POINTS * CIN, FILTERS) for W in (W_i, W_f, W_o, W_c)],
        axis=1)

    hf, cn = _comp_call(nf, xyz_p, kp_t, ch, w_flat)

    h = jnp.concatenate([xyz, hf], axis=-1)
    c = jnp.concatenate([xyz, cn], axis=-1)
    return h, c


# speedup vs baseline: 3.5351x; 3.5351x over previous
import functools

import jax
import jax.numpy as jnp
from jax import lax
from jax.experimental import pallas as pl
from jax.experimental.pallas import tpu as pltpu
from jax.experimental.pallas import tpu_sc as plsc

B, N = 4, 4096
C_FEAT = 64
FILTERS = 64
K_POINTS = 15
NSAMPLE = 16
CIN = C_FEAT + FILTERS
ROW = 128

BLKA = 128
TJ = 512
NT = N // TJ
BLKC = 128

_INF = 3.0e38
_BIGI = 2**30


def _knn_body(xyz_ref, xyzt_ref, cxyzt_ref, gidx_ref, nn_ref, nxyz_ref,
              d2_ref):
    b = pl.program_id(0)

    for t in range(NT):
        d2t = jnp.zeros((BLKA, TJ), jnp.float32)
        for c in range(3):
            xc = xyz_ref[0, :, c:c + 1]
            ac = xyzt_ref[0, c:c + 1, t * TJ:(t + 1) * TJ]
            d = xc - ac
            d2t = d2t + d * d
        d2_ref[:, t * TJ:(t + 1) * TJ] = d2t

    lv = jnp.full((BLKA, 1), -1.0, jnp.float32)
    li = jnp.full((BLKA, 1), -1, jnp.int32)
    cols = []
    for s in range(NSAMPLE):
        m = jnp.full((BLKA, 1), _INF, jnp.float32)
        mi = jnp.full((BLKA, 1), _BIGI, jnp.int32)
        for t in range(NT):
            d2t = d2_ref[:, t * TJ:(t + 1) * TJ]
            iot = lax.broadcasted_iota(jnp.int32, (BLKA, TJ), 1) + t * TJ
            mask = (d2t > lv) | ((d2t == lv) & (iot > li))
            vals = jnp.where(mask, d2t, _INF)
            mt = jnp.min(vals, axis=1, keepdims=True)
            it = jnp.min(jnp.where(vals == mt, iot, _BIGI), axis=1,
                         keepdims=True)
            better = (mt < m) | ((mt == m) & (it < mi))
            m = jnp.where(better, mt, m)
            mi = jnp.where(better, it, mi)
        lv, li = m, mi
        cols.append(mi)
        accs = [jnp.zeros((BLKA, 1), jnp.float32) for _ in range(3)]
        for t in range(NT):
            iot = lax.broadcasted_iota(jnp.int32, (BLKA, TJ), 1) + t * TJ
            oh = iot == mi
            for c in range(3):
                ac = xyzt_ref[0, c:c + 1, t * TJ:(t + 1) * TJ]
                accs[c] = accs[c] + jnp.sum(
                    jnp.where(oh, ac, 0.0), axis=1, keepdims=True)
        nxyz_ref[0, s] = jnp.concatenate(
            accs + [jnp.zeros((BLKA, 1), jnp.float32)], axis=1)
    gidx_ref[0] = jnp.concatenate(cols, axis=1) + b * N

    bm = jnp.full((BLKA, 1), _INF, jnp.float32)
    bi = jnp.full((BLKA, 1), _BIGI, jnp.int32)
    for t in range(NT):
        d2t = jnp.zeros((BLKA, TJ), jnp.float32)
        for c in range(3):
            xc = xyz_ref[0, :, c:c + 1]
            ac = cxyzt_ref[0, c:c + 1, t * TJ:(t + 1) * TJ]
            d = xc - ac
            d2t = d2t + d * d
        iot = lax.broadcasted_iota(jnp.int32, (BLKA, TJ), 1) + t * TJ
        mt = jnp.min(d2t, axis=1, keepdims=True)
        it = jnp.min(jnp.where(d2t == mt, iot, _BIGI), axis=1, keepdims=True)
        better = (mt < bm) | ((mt == bm) & (it < bi))
        bm = jnp.where(better, mt, bm)
        bi = jnp.where(better, it, bi)
    nn_ref[0] = bi + b * N


def _knn_call(xyz_p, xyz_t, cxyz_t):
    return pl.pallas_call(
        _knn_body,
        grid=(B, N // BLKA),
        in_specs=[
            pl.BlockSpec((1, BLKA, 4), lambda b, i: (b, i, 0)),
            pl.BlockSpec((1, 4, N), lambda b, i: (b, 0, 0)),
            pl.BlockSpec((1, 4, N), lambda b, i: (b, 0, 0)),
        ],
        out_specs=[
            pl.BlockSpec((1, BLKA, NSAMPLE), lambda b, i: (b, i, 0)),
            pl.BlockSpec((1, BLKA, 1), lambda b, i: (b, i, 0)),
            pl.BlockSpec((1, NSAMPLE, BLKA, 4), lambda b, i: (b, 0, i, 0)),
        ],
        out_shape=[
            jax.ShapeDtypeStruct((B, N, NSAMPLE), jnp.int32),
            jax.ShapeDtypeStruct((B, N, 1), jnp.int32),
            jax.ShapeDtypeStruct((B, NSAMPLE, N, 4), jnp.float32),
        ],
        scratch_shapes=[pltpu.VMEM((BLKA, N), jnp.float32)],
    )(xyz_p, xyz_t, cxyz_t)


_NW = 32
_CHUNK = 128
_FE = B * N * NSAMPLE
_FE_PER = _FE // _NW
_F_ITERS = _FE_PER // _CHUNK
_CH_PER = (B * N) // _NW
_C_ITERS = _CH_PER // _CHUNK


def _gather_body(tbl_hbm, gidx_hbm, c0f_hbm, nn_hbm, nf_out, ch_out,
                 idx_v, rows_v, rows2_v, sem):
    wid = lax.axis_index("s") * 2 + lax.axis_index("c")

    fbase = wid * _FE_PER

    def fbody(t, carry):
        off = fbase + t * _CHUNK
        pltpu.sync_copy(gidx_hbm.at[pl.ds(off, _CHUNK)], idx_v)
        pltpu.async_copy(tbl_hbm.at[idx_v], rows_v, sem).wait()
        pltpu.sync_copy(rows_v, nf_out.at[pl.ds(off, _CHUNK)])
        return carry

    lax.fori_loop(0, _F_ITERS, fbody, 0)

    cbase = wid * _CH_PER

    def cbody(t, carry):
        off = cbase + t * _CHUNK
        pltpu.sync_copy(nn_hbm.at[pl.ds(off, _CHUNK)], idx_v)
        pltpu.async_copy(c0f_hbm.at[idx_v], rows2_v, sem).wait()
        pltpu.sync_copy(rows2_v, ch_out.at[pl.ds(off, _CHUNK)])
        return carry

    lax.fori_loop(0, _C_ITERS, cbody, 0)


def _gather_call(tbl, gidx_f, c0f, nn_f):
    mesh = plsc.VectorSubcoreMesh(core_axis_name="c", subcore_axis_name="s")
    gk = functools.partial(
        pl.kernel,
        out_type=[
            jax.ShapeDtypeStruct((_FE, ROW), jnp.float32),
            jax.ShapeDtypeStruct((B * N, ROW), jnp.float32),
        ],
        mesh=mesh,
        scratch_types=[
            pltpu.VMEM((_CHUNK,), jnp.int32),
            pltpu.VMEM((_CHUNK, ROW), jnp.float32),
            pltpu.VMEM((_CHUNK, ROW), jnp.float32),
            pltpu.SemaphoreType.DMA,
        ],
    )(_gather_body)
    return gk(tbl, gidx_f, c0f, nn_f)


def _comp_body(nf_ref, nxyz_ref, xyz_ref, kp_ref, ch_ref, w_ref, hf_ref,
               cn_ref, wf_ref):
    xs = [xyz_ref[0, :, c:c + 1] for c in range(3)]
    for s in range(NSAMPLE):
        nx = nxyz_ref[0, s]
        dist2 = jnp.full((BLKC, K_POINTS), 1e-12, jnp.float32)
        for c in range(3):
            rel = nx[:, c:c + 1] - xs[c]
            d = rel - kp_ref[c:c + 1, :]
            dist2 = dist2 + d * d
        infl = jnp.maximum(1.0 - jnp.sqrt(dist2), 0.0)
        feat = nf_ref[0, s]
        for k in range(K_POINTS):
            w = infl[:, k:k + 1] * feat
            if s == 0:
                wf_ref[:, k * CIN:(k + 1) * CIN] = w
            else:
                wf_ref[:, k * CIN:(k + 1) * CIN] += w

    out = jnp.dot(wf_ref[...], w_ref[...],
                  preferred_element_type=jnp.float32)

    def hs(x):
        return jnp.clip(0.2 * x + 0.5, 0.0, 1.0)

    i_g = hs(out[:, 0:FILTERS])
    f_g = hs(out[:, FILTERS:2 * FILTERS])
    o_g = hs(out[:, 2 * FILTERS:3 * FILTERS])
    cc = out[:, 3 * FILTERS:4 * FILTERS]
    c_new = f_g * ch_ref[0, :, 0:FILTERS] + i_g * jnp.tanh(cc)
    hf_ref[0] = o_g * jnp.tanh(c_new)
    cn_ref[0] = c_new


def _comp_call(nf, nxyz, xyz_p, kp_t, ch, w_flat):
    return pl.pallas_call(
        _comp_body,
        grid=(B, N // BLKC),
        in_specs=[
            pl.BlockSpec((1, NSAMPLE, BLKC, ROW), lambda b, i: (b, 0, i, 0)),
            pl.BlockSpec((1, NSAMPLE, BLKC, 4), lambda b, i: (b, 0, i, 0)),
            pl.BlockSpec((1, BLKC, 4), lambda b, i: (b, i, 0)),
            pl.BlockSpec((3, K_POINTS), lambda b, i: (0, 0)),
            pl.BlockSpec((1, BLKC, ROW), lambda b, i: (b, i, 0)),
            pl.BlockSpec((K_POINTS * CIN, 4 * FILTERS), lambda b, i: (0, 0)),
        ],
        out_specs=[
            pl.BlockSpec((1, BLKC, FILTERS), lambda b, i: (b, i, 0)),
            pl.BlockSpec((1, BLKC, FILTERS), lambda b, i: (b, i, 0)),
        ],
        out_shape=[
            jax.ShapeDtypeStruct((B, N, FILTERS), jnp.float32),
            jax.ShapeDtypeStruct((B, N, FILTERS), jnp.float32),
        ],
        scratch_shapes=[pltpu.VMEM((BLKC, K_POINTS * CIN), jnp.float32)],
    )(nf, nxyz, xyz_p, kp_t, ch, w_flat)


def kernel(inputs, h0, c0, kernel_points, W_i, W_f, W_o, W_c):
    xyz = inputs[..., :3]
    feats = jnp.concatenate([h0[..., 3:], inputs[..., 3:]], axis=-1)

    zero1 = jnp.zeros((B, N, 1), jnp.float32)
    xyz_p = jnp.concatenate([xyz, zero1], axis=-1)
    xyz_t = jnp.concatenate(
        [jnp.transpose(xyz, (0, 2, 1)),
         jnp.zeros((B, 1, N), jnp.float32)], axis=1)
    cxyz_t = jnp.concatenate(
        [jnp.transpose(c0[..., :3], (0, 2, 1)),
         jnp.zeros((B, 1, N), jnp.float32)], axis=1)

    gidx, nn, nxyz = _knn_call(xyz_p, xyz_t, cxyz_t)

    tbl = feats.reshape(B * N, ROW)
    c0f = jnp.concatenate(
        [c0[..., 3:], jnp.zeros((B, N, ROW - FILTERS), jnp.float32)],
        axis=-1).reshape(B * N, ROW)

    gidx_t = jnp.transpose(gidx, (0, 2, 1))
    nf, ch = _gather_call(tbl, gidx_t.reshape(_FE), c0f, nn.reshape(B * N))
    nf = nf.reshape(B, NSAMPLE, N, ROW)
    ch = ch.reshape(B, N, ROW)

    kp_t = jnp.transpose(kernel_points, (1, 0))
    w_flat = jnp.concatenate(
        [W.reshape(K_POINTS * CIN, FILTERS) for W in (W_i, W_f, W_o, W_c)],
        axis=1)

    hf, cn = _comp_call(nf, nxyz, xyz_p, kp_t, ch, w_flat)

    h = jnp.concatenate([xyz, hf], axis=-1)
    c = jnp.concatenate([xyz, cn], axis=-1)
    return h, c


# speedup vs baseline: 8.0259x; 2.2704x over previous
import functools

import jax
import jax.numpy as jnp
from jax import lax
from jax.experimental import pallas as pl
from jax.experimental.pallas import tpu as pltpu
from jax.experimental.pallas import tpu_sc as plsc

B, N = 4, 4096
C_FEAT = 64
FILTERS = 64
K_POINTS = 15
NSAMPLE = 16
CIN = C_FEAT + FILTERS
ROW = 128

BLKA = 128
TJ = 512
NT = N // TJ
BLKC = 128

_INF = 3.0e38
_BIGI = 2**30


def _knn_body(xyz_ref, xyzt_ref, cxyzt_ref, g_ref, gidx_ref, nn_ref,
              nxyz_ref, d2_ref):
    b = pl.program_id(0)

    for t in range(NT):
        d2t = jnp.zeros((BLKA, TJ), jnp.float32)
        for c in range(3):
            xc = xyz_ref[0, :, c:c + 1]
            ac = xyzt_ref[0, c:c + 1, t * TJ:(t + 1) * TJ]
            d = xc - ac
            d2t = d2t + d * d
        d2_ref[:, t * TJ:(t + 1) * TJ] = d2t

    lv = jnp.full((BLKA, 1), -1.0, jnp.float32)
    cols = []
    for s in range(NSAMPLE + 1):
        m = jnp.full((BLKA, 1), _INF, jnp.float32)
        acc = jnp.zeros((BLKA, 8), jnp.float32)
        for t in range(NT):
            d2t = d2_ref[:, t * TJ:(t + 1) * TJ]
            if s > 0:
                oh = jnp.where(d2t == lv, 1.0, 0.0)
                acc = acc + jnp.dot(oh, g_ref[0, t * TJ:(t + 1) * TJ, :],
                                    preferred_element_type=jnp.float32)
            if s < NSAMPLE:
                vals = jnp.where(d2t > lv, d2t, _INF)
                m = jnp.minimum(m, jnp.min(vals, axis=1, keepdims=True))
        if s > 0:
            idx = jnp.clip(acc[:, 0:1].astype(jnp.int32), 0, N - 1)
            cols.append(idx)
            nxyz_ref[0, s - 1] = acc[:, 1:5]
        if s < NSAMPLE:
            lv = m
    gidx_ref[0] = jnp.concatenate(cols, axis=1) + b * N

    for t in range(NT):
        d2t = jnp.zeros((BLKA, TJ), jnp.float32)
        for c in range(3):
            xc = xyz_ref[0, :, c:c + 1]
            ac = cxyzt_ref[0, c:c + 1, t * TJ:(t + 1) * TJ]
            d = xc - ac
            d2t = d2t + d * d
        d2_ref[:, t * TJ:(t + 1) * TJ] = d2t
    bm = jnp.full((BLKA, 1), _INF, jnp.float32)
    for t in range(NT):
        bm = jnp.minimum(
            bm, jnp.min(d2_ref[:, t * TJ:(t + 1) * TJ], axis=1,
                        keepdims=True))
    acc = jnp.zeros((BLKA, 8), jnp.float32)
    for t in range(NT):
        oh = jnp.where(d2_ref[:, t * TJ:(t + 1) * TJ] == bm, 1.0, 0.0)
        acc = acc + jnp.dot(oh, g_ref[0, t * TJ:(t + 1) * TJ, :],
                            preferred_element_type=jnp.float32)
    nn_ref[0] = jnp.clip(acc[:, 0:1].astype(jnp.int32), 0, N - 1) + b * N


def _knn_call(xyz_p, xyz_t, cxyz_t, g_cols):
    return pl.pallas_call(
        _knn_body,
        grid=(B, N // BLKA),
        in_specs=[
            pl.BlockSpec((1, BLKA, 4), lambda b, i: (b, i, 0)),
            pl.BlockSpec((1, 4, N), lambda b, i: (b, 0, 0)),
            pl.BlockSpec((1, 4, N), lambda b, i: (b, 0, 0)),
            pl.BlockSpec((1, N, 8), lambda b, i: (b, 0, 0)),
        ],
        out_specs=[
            pl.BlockSpec((1, BLKA, NSAMPLE), lambda b, i: (b, i, 0)),
            pl.BlockSpec((1, BLKA, 1), lambda b, i: (b, i, 0)),
            pl.BlockSpec((1, NSAMPLE, BLKA, 4), lambda b, i: (b, 0, i, 0)),
        ],
        out_shape=[
            jax.ShapeDtypeStruct((B, N, NSAMPLE), jnp.int32),
            jax.ShapeDtypeStruct((B, N, 1), jnp.int32),
            jax.ShapeDtypeStruct((B, NSAMPLE, N, 4), jnp.float32),
        ],
        scratch_shapes=[pltpu.VMEM((BLKA, N), jnp.float32)],
    )(xyz_p, xyz_t, cxyz_t, g_cols)


_NW = 32
_CHUNK = 128
_FE = B * N * NSAMPLE
_FE_PER = _FE // _NW
_F_ITERS = _FE_PER // _CHUNK
_CH_PER = (B * N) // _NW
_C_ITERS = _CH_PER // _CHUNK


def _gather_body(tbl_hbm, gidx_hbm, c0f_hbm, nn_hbm, nf_out, ch_out,
                 idx_v, rows_v, rows2_v, sem):
    wid = lax.axis_index("s") * 2 + lax.axis_index("c")

    fbase = wid * _FE_PER

    def fbody(t, carry):
        off = fbase + t * _CHUNK
        pltpu.sync_copy(gidx_hbm.at[pl.ds(off, _CHUNK)], idx_v)
        pltpu.async_copy(tbl_hbm.at[idx_v], rows_v, sem).wait()
        pltpu.sync_copy(rows_v, nf_out.at[pl.ds(off, _CHUNK)])
        return carry

    lax.fori_loop(0, _F_ITERS, fbody, 0)

    cbase = wid * _CH_PER

    def cbody(t, carry):
        off = cbase + t * _CHUNK
        pltpu.sync_copy(nn_hbm.at[pl.ds(off, _CHUNK)], idx_v)
        pltpu.async_copy(c0f_hbm.at[idx_v], rows2_v, sem).wait()
        pltpu.sync_copy(rows2_v, ch_out.at[pl.ds(off, _CHUNK)])
        return carry

    lax.fori_loop(0, _C_ITERS, cbody, 0)


def _gather_call(tbl, gidx_f, c0f, nn_f):
    mesh = plsc.VectorSubcoreMesh(core_axis_name="c", subcore_axis_name="s")
    gk = functools.partial(
        pl.kernel,
        out_type=[
            jax.ShapeDtypeStruct((_FE, ROW), jnp.float32),
            jax.ShapeDtypeStruct((B * N, ROW), jnp.float32),
        ],
        mesh=mesh,
        scratch_types=[
            pltpu.VMEM((_CHUNK,), jnp.int32),
            pltpu.VMEM((_CHUNK, ROW), jnp.float32),
            pltpu.VMEM((_CHUNK, ROW), jnp.float32),
            pltpu.SemaphoreType.DMA,
        ],
    )(_gather_body)
    return gk(tbl, gidx_f, c0f, nn_f)


def _comp_body(nf_ref, nxyz_ref, xyz_ref, kp_ref, ch_ref, w_ref, hf_ref,
               cn_ref, wf_ref):
    xs = [xyz_ref[0, :, c:c + 1] for c in range(3)]
    for s in range(NSAMPLE):
        nx = nxyz_ref[0, s]
        dist2 = jnp.full((BLKC, K_POINTS), 1e-12, jnp.float32)
        for c in range(3):
            rel = nx[:, c:c + 1] - xs[c]
            d = rel - kp_ref[c:c + 1, :]
            dist2 = dist2 + d * d
        infl = jnp.maximum(1.0 - jnp.sqrt(dist2), 0.0)
        feat = nf_ref[0, s]
        for k in range(K_POINTS):
            w = infl[:, k:k + 1] * feat
            if s == 0:
                wf_ref[:, k * CIN:(k + 1) * CIN] = w
            else:
                wf_ref[:, k * CIN:(k + 1) * CIN] += w

    out = jnp.dot(wf_ref[...], w_ref[...],
                  preferred_element_type=jnp.float32)

    def hs(x):
        return jnp.clip(0.2 * x + 0.5, 0.0, 1.0)

    i_g = hs(out[:, 0:FILTERS])
    f_g = hs(out[:, FILTERS:2 * FILTERS])
    o_g = hs(out[:, 2 * FILTERS:3 * FILTERS])
    cc = out[:, 3 * FILTERS:4 * FILTERS]
    c_new = f_g * ch_ref[0, :, 0:FILTERS] + i_g * jnp.tanh(cc)
    hf_ref[0] = o_g * jnp.tanh(c_new)
    cn_ref[0] = c_new


def _comp_call(nf, nxyz, xyz_p, kp_t, ch, w_flat):
    return pl.pallas_call(
        _comp_body,
        grid=(B, N // BLKC),
        in_specs=[
            pl.BlockSpec((1, NSAMPLE, BLKC, ROW), lambda b, i: (b, 0, i, 0)),
            pl.BlockSpec((1, NSAMPLE, BLKC, 4), lambda b, i: (b, 0, i, 0)),
            pl.BlockSpec((1, BLKC, 4), lambda b, i: (b, i, 0)),
            pl.BlockSpec((3, K_POINTS), lambda b, i: (0, 0)),
            pl.BlockSpec((1, BLKC, ROW), lambda b, i: (b, i, 0)),
            pl.BlockSpec((K_POINTS * CIN, 4 * FILTERS), lambda b, i: (0, 0)),
        ],
        out_specs=[
            pl.BlockSpec((1, BLKC, FILTERS), lambda b, i: (b, i, 0)),
            pl.BlockSpec((1, BLKC, FILTERS), lambda b, i: (b, i, 0)),
        ],
        out_shape=[
            jax.ShapeDtypeStruct((B, N, FILTERS), jnp.float32),
            jax.ShapeDtypeStruct((B, N, FILTERS), jnp.float32),
        ],
        scratch_shapes=[pltpu.VMEM((BLKC, K_POINTS * CIN), jnp.float32)],
    )(nf, nxyz, xyz_p, kp_t, ch, w_flat)


def kernel(inputs, h0, c0, kernel_points, W_i, W_f, W_o, W_c):
    xyz = inputs[..., :3]
    feats = jnp.concatenate([h0[..., 3:], inputs[..., 3:]], axis=-1)

    zero1 = jnp.zeros((B, N, 1), jnp.float32)
    xyz_p = jnp.concatenate([xyz, zero1], axis=-1)
    xyz_t = jnp.concatenate(
        [jnp.transpose(xyz, (0, 2, 1)),
         jnp.zeros((B, 1, N), jnp.float32)], axis=1)
    cxyz_t = jnp.concatenate(
        [jnp.transpose(c0[..., :3], (0, 2, 1)),
         jnp.zeros((B, 1, N), jnp.float32)], axis=1)

    iota_col = jnp.broadcast_to(
        jnp.arange(N, dtype=jnp.float32)[None, :, None], (B, N, 1))
    g_cols = jnp.concatenate(
        [iota_col, xyz, jnp.zeros((B, N, 4), jnp.float32)], axis=-1)

    gidx, nn, nxyz = _knn_call(xyz_p, xyz_t, cxyz_t, g_cols)

    tbl = feats.reshape(B * N, ROW)
    c0f = jnp.concatenate(
        [c0[..., 3:], jnp.zeros((B, N, ROW - FILTERS), jnp.float32)],
        axis=-1).reshape(B * N, ROW)

    gidx_t = jnp.transpose(gidx, (0, 2, 1))
    nf, ch = _gather_call(tbl, gidx_t.reshape(_FE), c0f, nn.reshape(B * N))
    nf = nf.reshape(B, NSAMPLE, N, ROW)
    ch = ch.reshape(B, N, ROW)

    kp_t = jnp.transpose(kernel_points, (1, 0))
    w_flat = jnp.concatenate(
        [W.reshape(K_POINTS * CIN, FILTERS) for W in (W_i, W_f, W_o, W_c)],
        axis=1)

    hf, cn = _comp_call(nf, nxyz, xyz_p, kp_t, ch, w_flat)

    h = jnp.concatenate([xyz, hf], axis=-1)
    c = jnp.concatenate([xyz, cn], axis=-1)
    return h, c
